# Initial kernel scaffold; baseline (speedup 1.0000x reference)
import jax
import jax.numpy as jnp
from jax.experimental import pallas as pl


def kernel(products, users, product_embedding, user_embedding):
    raise NotImplementedError("write your pallas kernel here")


# speedup vs baseline: 1.9414x; 1.9414x over previous
import jax
import jax.numpy as jnp
from jax.experimental import pallas as pl
from jax.experimental.pallas import tpu as pltpu

_TN = 512
_CHUNK = 8


def _round_up(x, m):
    return ((x + m - 1) // m) * m


def _gather_dot_kernel(pids_ref, uids_ref, ptab_ref, utab_ref, out_ref, c_ref):
    tn = out_ref.shape[1]

    def chunk_body(c, carry):
        base = c * _CHUNK
        rows = []
        for i in range(_CHUNK):
            p = pids_ref[0, base + i]
            u = uids_ref[0, base + i]
            rows.append(ptab_ref[p] * utab_ref[u])
        blk = jnp.concatenate(rows, axis=0)
        c_ref[pl.ds(pl.multiple_of(base, _CHUNK), _CHUNK), :] = blk
        return carry

    jax.lax.fori_loop(0, tn // _CHUNK, chunk_body, 0)

    ones = jnp.ones((1, 128), jnp.float32)
    out_ref[...] = jax.lax.dot_general(
        ones, c_ref[...], (((1,), (1,)), ((), ())),
        preferred_element_type=jnp.float32)


def kernel(products, users, product_embedding, user_embedding):
    n = products.shape[0]
    p_rows, d = product_embedding.shape
    u_rows, d_u = user_embedding.shape
    assert d == d_u == 128

    n_pad = _round_up(n, _TN)
    num_tiles = n_pad // _TN

    def prep_ids(ids, rows):
        ids = jnp.clip(jnp.asarray(ids).astype(jnp.int32), 0, rows - 1)
        ids = jnp.pad(ids, (0, n_pad - n))
        return ids.reshape(num_tiles, 1, _TN)

    prod_ids = prep_ids(products, p_rows)
    user_ids = prep_ids(users, u_rows)

    ptab = product_embedding.astype(jnp.float32).reshape(p_rows, 1, d)
    utab = user_embedding.astype(jnp.float32).reshape(u_rows, 1, d)

    table_bytes = (p_rows + u_rows) * d * 4
    vmem_limit = min(int(2 * table_bytes + 4 * _TN * 128 * 4 + (8 << 20)),
                     60 << 20)

    cost = pl.CostEstimate(
        flops=2 * n_pad * d + 2 * n_pad * d,
        transcendentals=0,
        bytes_accessed=2 * n_pad * 4 + 2 * table_bytes + n_pad * 4,
    )

    out = pl.pallas_call(
        _gather_dot_kernel,
        out_shape=jax.ShapeDtypeStruct((num_tiles, 1, _TN), jnp.float32),
        grid=(num_tiles,),
        in_specs=[
            pl.BlockSpec((None, 1, _TN), lambda t: (t, 0, 0),
                         memory_space=pltpu.SMEM),
            pl.BlockSpec((None, 1, _TN), lambda t: (t, 0, 0),
                         memory_space=pltpu.SMEM),
            pl.BlockSpec((p_rows, 1, d), lambda t: (0, 0, 0)),
            pl.BlockSpec((u_rows, 1, d), lambda t: (0, 0, 0)),
        ],
        out_specs=pl.BlockSpec((None, 1, _TN), lambda t: (t, 0, 0)),
        scratch_shapes=[pltpu.VMEM((_TN, d), jnp.float32)],
        compiler_params=pltpu.CompilerParams(
            dimension_semantics=("parallel",),
            vmem_limit_bytes=vmem_limit,
        ),
        cost_estimate=cost,
    )(prod_ids, user_ids, ptab, utab)
    return out.reshape(n_pad)[:n]
